# baseline (speedup 1.0000x reference)
import functools

import jax
import jax.numpy as jnp
from jax import lax
from jax.experimental import pallas as pl
from jax.experimental.pallas import tpu as pltpu
from jax.experimental.pallas import tpu_sc as plsc

NC = 2
NS = 16
NW = NC * NS
CHUNK = 128
SUP = 8


def _sc_features(x, src_slab, dst_slab, n_chunks, n_acc, d):
    stripe = n_acc // NS
    mesh = plsc.VectorSubcoreMesh(core_axis_name="c", subcore_axis_name="s")

    n_sup = n_chunks // SUP

    @functools.partial(
        pl.kernel,
        out_type=jax.ShapeDtypeStruct((NC, n_acc, d), jnp.float32),
        mesh=mesh,
        scratch_types=[
            pltpu.VMEM((SUP, CHUNK), jnp.int32),
            pltpu.VMEM((SUP, CHUNK), jnp.int32),
            pltpu.VMEM((CHUNK, d), jnp.float32),
            pltpu.VMEM((CHUNK, d), jnp.float32),
            pltpu.VMEM_SHARED((n_acc, d), jnp.float32),
            pltpu.SemaphoreType.DMA,
            pltpu.SemaphoreType.DMA,
        ],
    )
    def agg(x_hbm, src_hbm, dst_hbm, acc_out,
            src_s, dst_s, rows0, rows1, acc_sh, sem0, sem1):
        c = lax.axis_index("c")
        s = lax.axis_index("s")
        wid = c * NS + s
        zero16 = jnp.zeros((16,), jnp.float32)

        def init_row(i, _):
            for k in range(d // 16):
                rows0[i, pl.ds(16 * k, 16)] = zero16
            return 0
        lax.fori_loop(0, CHUNK, init_row, 0)

        for t in range(stripe // CHUNK):
            pltpu.sync_copy(rows0, acc_sh.at[pl.ds(s * stripe + t * CHUNK, CHUNK)])
        plsc.subcore_barrier()

        bufs = (rows0, rows1)
        sems = (sem0, sem1)

        def sup_body(g, _):
            pltpu.sync_copy(src_hbm.at[wid, pl.ds(g * SUP, SUP)], src_s)
            pltpu.sync_copy(dst_hbm.at[wid, pl.ds(g * SUP, SUP)], dst_s)
            descs = [None, None]
            descs[0] = pltpu.async_copy(x_hbm.at[src_s.at[0]], rows0, sem0)
            for k in range(SUP):
                if k + 1 < SUP:
                    nb = (k + 1) % 2
                    descs[nb] = pltpu.async_copy(
                        x_hbm.at[src_s.at[k + 1]], bufs[nb], sems[nb])
                descs[k % 2].wait()
                pltpu.sync_copy(bufs[k % 2], acc_sh.at[dst_s.at[k]], add=True)
            return 0
        lax.fori_loop(0, n_sup, sup_body, 0)
        plsc.subcore_barrier()

        for t in range(stripe // CHUNK):
            base = s * stripe + t * CHUNK
            pltpu.sync_copy(acc_sh.at[pl.ds(base, CHUNK)], rows0)
            pltpu.sync_copy(rows0, acc_out.at[c, pl.ds(base, CHUNK)])

    return agg(x, src_slab, dst_slab)


def _sc_degrees(dst_slab, n_chunks, n_acc, d):
    stripe = n_acc // NS
    mesh = plsc.VectorSubcoreMesh(core_axis_name="c", subcore_axis_name="s")

    n_sup = n_chunks // SUP

    @functools.partial(
        pl.kernel,
        out_type=jax.ShapeDtypeStruct((NC, n_acc, d), jnp.float32),
        mesh=mesh,
        scratch_types=[
            pltpu.VMEM((SUP, CHUNK), jnp.int32),
            pltpu.VMEM((CHUNK, d), jnp.float32),
            pltpu.VMEM_SHARED((n_acc, d), jnp.float32),
            pltpu.SemaphoreType.DMA,
        ],
    )
    def deg(dst_hbm, deg_out, dst_s, e0_v, deg_sh, sem):
        c = lax.axis_index("c")
        s = lax.axis_index("s")
        wid = c * NS + s
        zero16 = jnp.zeros((16,), jnp.float32)
        e0 = jnp.where(lax.iota(jnp.int32, 16) == 0, 1.0, 0.0)

        def init_zero(i, _):
            for k in range(d // 16):
                e0_v[i, pl.ds(16 * k, 16)] = zero16
            return 0
        lax.fori_loop(0, CHUNK, init_zero, 0)
        for t in range(stripe // CHUNK):
            pltpu.sync_copy(e0_v, deg_sh.at[pl.ds(s * stripe + t * CHUNK, CHUNK)])

        def init_e0(i, _):
            e0_v[i, pl.ds(0, 16)] = e0
            return 0
        lax.fori_loop(0, CHUNK, init_e0, 0)
        plsc.subcore_barrier()

        def sup_body(g, _):
            pltpu.sync_copy(dst_hbm.at[wid, pl.ds(g * SUP, SUP)], dst_s)
            descs = [
                pltpu.async_copy(e0_v, deg_sh.at[dst_s.at[k]], sem, add=True)
                for k in range(SUP)
            ]
            for dsc in descs:
                dsc.wait()
            return 0
        lax.fori_loop(0, n_sup, sup_body, 0)
        plsc.subcore_barrier()

        for t in range(stripe // CHUNK):
            base = s * stripe + t * CHUNK
            pltpu.sync_copy(deg_sh.at[pl.ds(base, CHUNK)], e0_v)
            pltpu.sync_copy(e0_v, deg_out.at[c, pl.ds(base, CHUNK)])

    return deg(dst_slab)


def _tc_body(acc_ref, deg_ref, x_ref, w_ref, rw_ref, b_ref, o_ref):
    summed = acc_ref[0] + acc_ref[1]
    deg = jnp.sum(deg_ref[0] + deg_ref[1], axis=1, keepdims=True)
    mean = summed / jnp.maximum(deg, 1.0)
    o_ref[...] = (
        jnp.dot(mean, w_ref[...], preferred_element_type=jnp.float32)
        + jnp.dot(x_ref[...], rw_ref[...], preferred_element_type=jnp.float32)
        + b_ref[...]
    )


def kernel(x, edge_index, weight, root_weight, bias):
    n, d = x.shape
    e = edge_index.shape[1]

    epw = -(-e // NW)
    n_chunks = -(-epw // (CHUNK * SUP)) * SUP
    e_pad = NW * n_chunks * CHUNK

    n_acc = -(-(n + 1) // (NS * CHUNK)) * (NS * CHUNK)

    src = edge_index[0].astype(jnp.int32)
    dst = edge_index[1].astype(jnp.int32)
    pad = e_pad - e
    src_slab = jnp.concatenate([src, jnp.zeros((pad,), jnp.int32)])
    dst_slab = jnp.concatenate([dst, jnp.full((pad,), n_acc - 1, jnp.int32)])
    src_slab = src_slab.reshape(NW, n_chunks, CHUNK)
    dst_slab = dst_slab.reshape(NW, n_chunks, CHUNK)

    acc = _sc_features(x, src_slab, dst_slab, n_chunks, n_acc, d)
    deg = _sc_degrees(dst_slab, n_chunks, n_acc, d)

    xp = jnp.concatenate([x, jnp.zeros((n_acc - n, d), jnp.float32)])
    bias2 = bias.reshape(1, d)

    blk = 512
    out = pl.pallas_call(
        _tc_body,
        grid=(n_acc // blk,),
        in_specs=[
            pl.BlockSpec((NC, blk, d), lambda i: (0, i, 0)),
            pl.BlockSpec((NC, blk, d), lambda i: (0, i, 0)),
            pl.BlockSpec((blk, d), lambda i: (i, 0)),
            pl.BlockSpec((d, d), lambda i: (0, 0)),
            pl.BlockSpec((d, d), lambda i: (0, 0)),
            pl.BlockSpec((1, d), lambda i: (0, 0)),
        ],
        out_specs=pl.BlockSpec((blk, d), lambda i: (i, 0)),
        out_shape=jax.ShapeDtypeStruct((n_acc, d), jnp.float32),
    )(acc, deg, xp, weight, root_weight, bias2)

    return out[:n]


# speedup vs baseline: 1.2115x; 1.2115x over previous
import functools

import jax
import jax.numpy as jnp
from jax import lax
from jax.experimental import pallas as pl
from jax.experimental.pallas import tpu as pltpu
from jax.experimental.pallas import tpu_sc as plsc

NC = 2
NS = 16
NW = NC * NS
CHUNK = 128
SUP = 8


def _sc_features(x, src_slab, dst_slab, n_chunks, n_acc, d):
    stripe = n_acc // NS
    mesh = plsc.VectorSubcoreMesh(core_axis_name="c", subcore_axis_name="s")

    @functools.partial(
        pl.kernel,
        out_type=jax.ShapeDtypeStruct((NC, n_acc, d), jnp.float32),
        mesh=mesh,
        scratch_types=[
            pltpu.VMEM((CHUNK,), jnp.int32),
            pltpu.VMEM((CHUNK,), jnp.int32),
            pltpu.VMEM((CHUNK, d), jnp.float32),
            pltpu.VMEM_SHARED((n_acc, d), jnp.float32),
            pltpu.SemaphoreType.DMA,
        ],
    )
    def agg(x_hbm, src_hbm, dst_hbm, acc_out, src_c, dst_c, rows_v, acc_sh, sem):
        c = lax.axis_index("c")
        s = lax.axis_index("s")
        wid = c * NS + s
        zero16 = jnp.zeros((16,), jnp.float32)

        def init_row(i, _):
            for k in range(d // 16):
                rows_v[i, pl.ds(16 * k, 16)] = zero16
            return 0
        lax.fori_loop(0, CHUNK, init_row, 0)

        for t in range(stripe // CHUNK):
            pltpu.sync_copy(rows_v, acc_sh.at[pl.ds(s * stripe + t * CHUNK, CHUNK)])
        plsc.subcore_barrier()

        def body(j, _):
            pltpu.sync_copy(src_hbm.at[wid, j], src_c)
            pltpu.sync_copy(dst_hbm.at[wid, j], dst_c)
            pltpu.async_copy(x_hbm.at[src_c], rows_v, sem).wait()
            pltpu.sync_copy(rows_v, acc_sh.at[dst_c], add=True)
            return 0
        lax.fori_loop(0, n_chunks, body, 0)
        plsc.subcore_barrier()

        for t in range(stripe // CHUNK):
            base = s * stripe + t * CHUNK
            pltpu.sync_copy(acc_sh.at[pl.ds(base, CHUNK)], rows_v)
            pltpu.sync_copy(rows_v, acc_out.at[c, pl.ds(base, CHUNK)])

    return agg(x, src_slab, dst_slab)


def _sc_degrees(dst_slab, n_chunks, n_acc, d):
    stripe = n_acc // NS
    mesh = plsc.VectorSubcoreMesh(core_axis_name="c", subcore_axis_name="s")

    n_sup = n_chunks // SUP

    @functools.partial(
        pl.kernel,
        out_type=jax.ShapeDtypeStruct((NC, n_acc, d), jnp.float32),
        mesh=mesh,
        scratch_types=[
            pltpu.VMEM((SUP, CHUNK), jnp.int32),
            pltpu.VMEM((CHUNK, d), jnp.float32),
            pltpu.VMEM_SHARED((n_acc, d), jnp.float32),
            pltpu.SemaphoreType.DMA,
        ],
    )
    def deg(dst_hbm, deg_out, dst_s, e0_v, deg_sh, sem):
        c = lax.axis_index("c")
        s = lax.axis_index("s")
        wid = c * NS + s
        zero16 = jnp.zeros((16,), jnp.float32)
        e0 = jnp.where(lax.iota(jnp.int32, 16) == 0, 1.0, 0.0)

        def init_zero(i, _):
            for k in range(d // 16):
                e0_v[i, pl.ds(16 * k, 16)] = zero16
            return 0
        lax.fori_loop(0, CHUNK, init_zero, 0)
        for t in range(stripe // CHUNK):
            pltpu.sync_copy(e0_v, deg_sh.at[pl.ds(s * stripe + t * CHUNK, CHUNK)])

        def init_e0(i, _):
            e0_v[i, pl.ds(0, 16)] = e0
            return 0
        lax.fori_loop(0, CHUNK, init_e0, 0)
        plsc.subcore_barrier()

        def sup_body(g, _):
            pltpu.sync_copy(dst_hbm.at[wid, pl.ds(g * SUP, SUP)], dst_s)
            descs = [
                pltpu.async_copy(e0_v, deg_sh.at[dst_s.at[k]], sem, add=True)
                for k in range(SUP)
            ]
            for dsc in descs:
                dsc.wait()
            return 0
        lax.fori_loop(0, n_sup, sup_body, 0)
        plsc.subcore_barrier()

        for t in range(stripe // CHUNK):
            base = s * stripe + t * CHUNK
            pltpu.sync_copy(deg_sh.at[pl.ds(base, CHUNK)], e0_v)
            pltpu.sync_copy(e0_v, deg_out.at[c, pl.ds(base, CHUNK)])

    return deg(dst_slab)


def _tc_body(acc_ref, deg_ref, x_ref, w_ref, rw_ref, b_ref, o_ref):
    summed = acc_ref[0] + acc_ref[1]
    deg = jnp.sum(deg_ref[0] + deg_ref[1], axis=1, keepdims=True)
    mean = summed / jnp.maximum(deg, 1.0)
    o_ref[...] = (
        jnp.dot(mean, w_ref[...], preferred_element_type=jnp.float32)
        + jnp.dot(x_ref[...], rw_ref[...], preferred_element_type=jnp.float32)
        + b_ref[...]
    )


def kernel(x, edge_index, weight, root_weight, bias):
    n, d = x.shape
    e = edge_index.shape[1]

    n_acc = -(-(n + 1) // (NS * CHUNK)) * (NS * CHUNK)

    src = edge_index[0].astype(jnp.int32)
    dst = edge_index[1].astype(jnp.int32)

    n_chunks_f = -(-e // (NW * CHUNK))
    e_pad_f = NW * n_chunks_f * CHUNK
    pad_f = e_pad_f - e
    src_f = jnp.concatenate([src, jnp.zeros((pad_f,), jnp.int32)])
    dst_f = jnp.concatenate([dst, jnp.full((pad_f,), n_acc - 1, jnp.int32)])
    src_f = src_f.reshape(NW, n_chunks_f, CHUNK)
    dst_f = dst_f.reshape(NW, n_chunks_f, CHUNK)

    n_chunks_d = -(-e // (NW * CHUNK * SUP)) * SUP
    e_pad_d = NW * n_chunks_d * CHUNK
    pad_d = e_pad_d - e
    dst_d = jnp.concatenate([dst, jnp.full((pad_d,), n_acc - 1, jnp.int32)])
    dst_d = dst_d.reshape(NW, n_chunks_d, CHUNK)

    acc = _sc_features(x, src_f, dst_f, n_chunks_f, n_acc, d)
    deg = _sc_degrees(dst_d, n_chunks_d, n_acc, d)

    xp = jnp.concatenate([x, jnp.zeros((n_acc - n, d), jnp.float32)])
    bias2 = bias.reshape(1, d)

    blk = 512
    out = pl.pallas_call(
        _tc_body,
        grid=(n_acc // blk,),
        in_specs=[
            pl.BlockSpec((NC, blk, d), lambda i: (0, i, 0)),
            pl.BlockSpec((NC, blk, d), lambda i: (0, i, 0)),
            pl.BlockSpec((blk, d), lambda i: (i, 0)),
            pl.BlockSpec((d, d), lambda i: (0, 0)),
            pl.BlockSpec((d, d), lambda i: (0, 0)),
            pl.BlockSpec((1, d), lambda i: (0, 0)),
        ],
        out_specs=pl.BlockSpec((blk, d), lambda i: (i, 0)),
        out_shape=jax.ShapeDtypeStruct((n_acc, d), jnp.float32),
    )(acc, deg, xp, weight, root_weight, bias2)

    return out[:n]


# speedup vs baseline: 1.2123x; 1.0007x over previous
import functools

import jax
import jax.numpy as jnp
from jax import lax
from jax.experimental import pallas as pl
from jax.experimental.pallas import tpu as pltpu
from jax.experimental.pallas import tpu_sc as plsc

NC = 2
NS = 16
NW = NC * NS
CHUNK = 128
SUP = 8


def _sc_features(x, src_slab, dst_slab, n_chunks, n_acc, d):
    stripe = n_acc // NS
    mesh = plsc.VectorSubcoreMesh(core_axis_name="c", subcore_axis_name="s")

    @functools.partial(
        pl.kernel,
        out_type=jax.ShapeDtypeStruct((NC, n_acc, d), jnp.float32),
        mesh=mesh,
        scratch_types=[
            pltpu.VMEM((CHUNK,), jnp.int32),
            pltpu.VMEM((CHUNK,), jnp.int32),
            pltpu.VMEM((CHUNK, d), jnp.float32),
            pltpu.VMEM_SHARED((n_acc, d), jnp.float32),
            pltpu.SemaphoreType.DMA,
        ],
    )
    def agg(x_hbm, src_hbm, dst_hbm, acc_out, src_c, dst_c, rows_v, acc_sh, sem):
        c = lax.axis_index("c")
        s = lax.axis_index("s")
        wid = c * NS + s
        zero16 = jnp.zeros((16,), jnp.float32)

        def init_row(i, _):
            for k in range(d // 16):
                rows_v[i, pl.ds(16 * k, 16)] = zero16
            return 0
        lax.fori_loop(0, CHUNK, init_row, 0)

        for t in range(stripe // CHUNK):
            pltpu.sync_copy(rows_v, acc_sh.at[pl.ds(s * stripe + t * CHUNK, CHUNK)])
        plsc.subcore_barrier()

        def body(j, _):
            pltpu.sync_copy(src_hbm.at[wid, j], src_c)
            pltpu.sync_copy(dst_hbm.at[wid, j], dst_c)
            pltpu.async_copy(x_hbm.at[src_c], rows_v, sem).wait()
            pltpu.sync_copy(rows_v, acc_sh.at[dst_c], add=True)
            return 0
        lax.fori_loop(0, n_chunks, body, 0)
        plsc.subcore_barrier()

        for t in range(stripe // CHUNK):
            base = s * stripe + t * CHUNK
            pltpu.sync_copy(acc_sh.at[pl.ds(base, CHUNK)], rows_v)
            pltpu.sync_copy(rows_v, acc_out.at[c, pl.ds(base, CHUNK)])

    return agg(x, src_slab, dst_slab)


def _sc_degrees(dst_slab, n_chunks, n_acc, d):
    stripe = n_acc // NS
    mesh = plsc.VectorSubcoreMesh(core_axis_name="c", subcore_axis_name="s")

    n_sup = n_chunks // SUP

    @functools.partial(
        pl.kernel,
        out_type=jax.ShapeDtypeStruct((NC, n_acc, d), jnp.float32),
        mesh=mesh,
        scratch_types=[
            pltpu.VMEM((SUP, CHUNK), jnp.int32),
            pltpu.VMEM((CHUNK, d), jnp.float32),
            pltpu.VMEM_SHARED((n_acc, d), jnp.float32),
            pltpu.SemaphoreType.DMA,
        ],
    )
    def deg(dst_hbm, deg_out, dst_s, e0_v, deg_sh, sem):
        c = lax.axis_index("c")
        s = lax.axis_index("s")
        wid = c * NS + s
        zero16 = jnp.zeros((16,), jnp.float32)
        e0 = jnp.where(lax.iota(jnp.int32, 16) == 0, 1.0, 0.0)

        def init_zero(i, _):
            for k in range(d // 16):
                e0_v[i, pl.ds(16 * k, 16)] = zero16
            return 0
        lax.fori_loop(0, CHUNK, init_zero, 0)
        for t in range(stripe // CHUNK):
            pltpu.sync_copy(e0_v, deg_sh.at[pl.ds(s * stripe + t * CHUNK, CHUNK)])

        def init_e0(i, _):
            e0_v[i, pl.ds(0, 16)] = e0
            return 0
        lax.fori_loop(0, CHUNK, init_e0, 0)
        plsc.subcore_barrier()

        def sup_body(g, _):
            pltpu.sync_copy(dst_hbm.at[wid, pl.ds(g * SUP, SUP)], dst_s)
            descs = [
                pltpu.async_copy(e0_v, deg_sh.at[dst_s.at[k]], sem, add=True)
                for k in range(SUP)
            ]
            for dsc in descs:
                dsc.wait()
            return 0
        lax.fori_loop(0, n_sup, sup_body, 0)
        plsc.subcore_barrier()

        for t in range(stripe // CHUNK):
            base = s * stripe + t * CHUNK
            pltpu.sync_copy(deg_sh.at[pl.ds(base, CHUNK)], e0_v)
            pltpu.sync_copy(e0_v, deg_out.at[c, pl.ds(base, CHUNK)])

    return deg(dst_slab)


def _tc_root_body(x_ref, rw_ref, b_ref, o_ref):
    o_ref[...] = (
        jnp.dot(x_ref[...], rw_ref[...], preferred_element_type=jnp.float32)
        + b_ref[...]
    )


def _tc_body(acc_ref, deg_ref, root_ref, w_ref, o_ref):
    summed = acc_ref[0] + acc_ref[1]
    deg = jnp.sum(deg_ref[0] + deg_ref[1], axis=1, keepdims=True)
    mean = summed / jnp.maximum(deg, 1.0)
    o_ref[...] = (
        jnp.dot(mean, w_ref[...], preferred_element_type=jnp.float32)
        + root_ref[...]
    )


def kernel(x, edge_index, weight, root_weight, bias):
    n, d = x.shape
    e = edge_index.shape[1]

    n_acc = -(-(n + 1) // (NS * CHUNK)) * (NS * CHUNK)

    src = edge_index[0].astype(jnp.int32)
    dst = edge_index[1].astype(jnp.int32)

    n_chunks_f = -(-e // (NW * CHUNK))
    e_pad_f = NW * n_chunks_f * CHUNK
    pad_f = e_pad_f - e
    src_f = jnp.concatenate([src, jnp.zeros((pad_f,), jnp.int32)])
    dst_f = jnp.concatenate([dst, jnp.full((pad_f,), n_acc - 1, jnp.int32)])
    src_f = src_f.reshape(NW, n_chunks_f, CHUNK)
    dst_f = dst_f.reshape(NW, n_chunks_f, CHUNK)

    n_chunks_d = -(-e // (NW * CHUNK * SUP)) * SUP
    e_pad_d = NW * n_chunks_d * CHUNK
    pad_d = e_pad_d - e
    dst_d = jnp.concatenate([dst, jnp.full((pad_d,), n_acc - 1, jnp.int32)])
    dst_d = dst_d.reshape(NW, n_chunks_d, CHUNK)

    acc = _sc_features(x, src_f, dst_f, n_chunks_f, n_acc, d)
    deg = _sc_degrees(dst_d, n_chunks_d, n_acc, d)

    xp = jnp.concatenate([x, jnp.zeros((n_acc - n, d), jnp.float32)])
    bias2 = bias.reshape(1, d)

    blk = 512
    root = pl.pallas_call(
        _tc_root_body,
        grid=(n_acc // blk,),
        in_specs=[
            pl.BlockSpec((blk, d), lambda i: (i, 0)),
            pl.BlockSpec((d, d), lambda i: (0, 0)),
            pl.BlockSpec((1, d), lambda i: (0, 0)),
        ],
        out_specs=pl.BlockSpec((blk, d), lambda i: (i, 0)),
        out_shape=jax.ShapeDtypeStruct((n_acc, d), jnp.float32),
    )(xp, root_weight, bias2)

    out = pl.pallas_call(
        _tc_body,
        grid=(n_acc // blk,),
        in_specs=[
            pl.BlockSpec((NC, blk, d), lambda i: (0, i, 0)),
            pl.BlockSpec((NC, blk, d), lambda i: (0, i, 0)),
            pl.BlockSpec((blk, d), lambda i: (i, 0)),
            pl.BlockSpec((d, d), lambda i: (0, 0)),
        ],
        out_specs=pl.BlockSpec((blk, d), lambda i: (i, 0)),
        out_shape=jax.ShapeDtypeStruct((n_acc, d), jnp.float32),
    )(acc, deg, root, weight)

    return out[:n]
